# Initial kernel scaffold; baseline (speedup 1.0000x reference)
import jax
import jax.numpy as jnp
from jax.experimental import pallas as pl


def kernel(inputs, hidden_state, adj, W_gate_0, b_gate_0, W_cand_0, b_cand_0, W_gate_1, b_gate_1, W_cand_1, b_cand_1, W_proj, b_proj):
    raise NotImplementedError("write your pallas kernel here")


# speedup vs baseline: 2.8234x; 2.8234x over previous
import functools

import jax
import jax.numpy as jnp
from jax.experimental import pallas as pl
from jax.experimental.pallas import tpu as pltpu

N = 512
U = 64
B = 64
M = 5
F32 = jnp.float32
BF16 = jnp.bfloat16


def _mm(a, b):
    return jax.lax.dot_general(
        a.astype(BF16), b.astype(BF16),
        (((1,), (0,)), ((), ())), preferred_element_type=F32)


def _mm_t(a, b):
    return jax.lax.dot_general(
        a.astype(BF16), b.astype(BF16),
        (((0,), (0,)), ((), ())), preferred_element_type=F32)


def _dcgru_kernel(xT_ref, h0_ref, h1_ref, adj_ref,
                  wg0_ref, bg0_ref, wc0_ref, bc0_ref,
                  wg1_ref, bg1_ref, wc1_ref, bc1_ref,
                  wp_ref, bp_ref,
                  outT_ref, h0o_ref, h1o_ref,
                  a1_ref, a2_ref):
    b = pl.program_id(0)

    @pl.when(b == 0)
    def _build_supports():
        a = adj_ref[...]
        d = jnp.sum(a, axis=1, keepdims=True)
        a1_ref[...] = (a * jnp.where(d > 0, 1.0 / d, 0.0)).astype(BF16)
        c = jnp.sum(a, axis=0, keepdims=True)
        a2_ref[...] = (a * jnp.where(c > 0, 1.0 / c, 0.0)).astype(BF16)

    a1 = a1_ref[...]
    a2 = a2_ref[...]

    def diffuse(x0):
        t1 = _mm_t(a1, x0)
        t2 = 2.0 * _mm_t(a1, t1) - x0
        t3 = _mm(a2, x0)
        t4 = 2.0 * _mm(a2, t3) - x0
        return jnp.concatenate([x0, t1, t2, t3, t4], axis=1)

    def gconv(xs, w_ref, b_ref):
        return _mm(diffuse(xs), w_ref[...]) + b_ref[...]

    x = xT_ref[0]
    h0 = h0_ref[0]
    h1 = h1_ref[0]

    xs = jnp.concatenate([x, h0], axis=1)
    g = jax.nn.sigmoid(gconv(xs, wg0_ref, bg0_ref))
    r = g[:, :U]
    u = g[:, U:]
    xs2 = jnp.concatenate([x, r * h0], axis=1)
    cv = jnp.tanh(gconv(xs2, wc0_ref, bc0_ref))
    h0n = u * h0 + (1.0 - u) * cv
    h0o_ref[0] = h0n

    xs = jnp.concatenate([h0n, h1], axis=1)
    g = jax.nn.sigmoid(gconv(xs, wg1_ref, bg1_ref))
    r = g[:, :U]
    u = g[:, U:]
    xs2 = jnp.concatenate([h0n, r * h1], axis=1)
    cv = jnp.tanh(gconv(xs2, wc1_ref, bc1_ref))
    h1n = u * h1 + (1.0 - u) * cv
    h1o_ref[0] = h1n

    outT_ref[0] = _mm(h1n, wp_ref[...]) + bp_ref[...]


def _reorder(w, d):
    return w.reshape(d, M, -1).transpose(1, 0, 2).reshape(d * M, -1)


@functools.partial(jax.jit, static_argnames=())
def kernel(inputs, hidden_state, adj, W_gate_0, b_gate_0, W_cand_0, b_cand_0,
           W_gate_1, b_gate_1, W_cand_1, b_cand_1, W_proj, b_proj):
    xb = inputs.reshape(B, N, 1)
    h0 = hidden_state[0].reshape(B, N, U)
    h1 = hidden_state[1].reshape(B, N, U)
    wg0 = _reorder(W_gate_0, 1 + U)
    wc0 = _reorder(W_cand_0, 1 + U)
    wg1 = _reorder(W_gate_1, 2 * U)
    wc1 = _reorder(W_cand_1, 2 * U)

    full = lambda *shape: pl.BlockSpec(shape, lambda b: (0,) * len(shape))
    grid = (B,)
    outT, h0o, h1o = pl.pallas_call(
        _dcgru_kernel,
        grid=grid,
        in_specs=[
            pl.BlockSpec((1, N, 1), lambda b: (b, 0, 0)),
            pl.BlockSpec((1, N, U), lambda b: (b, 0, 0)),
            pl.BlockSpec((1, N, U), lambda b: (b, 0, 0)),
            full(N, N),
            full((1 + U) * M, 2 * U), full(1, 2 * U),
            full((1 + U) * M, U), full(1, U),
            full(2 * U * M, 2 * U), full(1, 2 * U),
            full(2 * U * M, U), full(1, U),
            full(U, 1), full(1, 1),
        ],
        out_specs=[
            pl.BlockSpec((1, N, 1), lambda b: (b, 0, 0)),
            pl.BlockSpec((1, N, U), lambda b: (b, 0, 0)),
            pl.BlockSpec((1, N, U), lambda b: (b, 0, 0)),
        ],
        out_shape=[
            jax.ShapeDtypeStruct((B, N, 1), F32),
            jax.ShapeDtypeStruct((B, N, U), F32),
            jax.ShapeDtypeStruct((B, N, U), F32),
        ],
        scratch_shapes=[
            pltpu.VMEM((N, N), BF16),
            pltpu.VMEM((N, N), BF16),
        ],
    )(xb, h0, h1, adj,
      wg0, b_gate_0.reshape(1, -1), wc0, b_cand_0.reshape(1, -1),
      wg1, b_gate_1.reshape(1, -1), wc1, b_cand_1.reshape(1, -1),
      W_proj, b_proj.reshape(1, 1))

    output = outT.reshape(B, N)
    hs = jnp.stack([h0o.reshape(B, N * U), h1o.reshape(B, N * U)])
    return (output, hs)


# speedup vs baseline: 4.2154x; 1.4930x over previous
import jax
import jax.numpy as jnp
from jax.experimental import pallas as pl
from jax.experimental.pallas import tpu as pltpu

N = 512
U = 64
B = 64
BC = 8
M = 5
F32 = jnp.float32
BF16 = jnp.bfloat16


def _dot(a, b, ca, cb):
    return jax.lax.dot_general(
        a.astype(BF16), b.astype(BF16),
        (((ca,), (cb,)), ((), ())), preferred_element_type=F32)


def _mm(a, b):
    return _dot(a, b, 1, 0)


def _dcgru_kernel(xt_ref, h0_ref, h1_ref, adj_ref,
                  kg_ref, wh0g_ref, bg0_ref, kc_ref, wh0c_ref, bc0_ref,
                  wa1g_ref, wb1g_ref, bg1_ref, wa1c_ref, wb1c_ref, bc1_ref,
                  wpbd_ref, bp_ref,
                  out_ref, h0o_ref, h1o_ref,
                  a1_ref, a2_ref, gx0_ref, cx0_ref):
    s = pl.program_id(0)

    @pl.when(s == 0)
    def _prologue():
        a = adj_ref[...]
        d = jnp.sum(a, axis=1, keepdims=True)
        a1_ref[...] = (a * jnp.where(d > 0, 1.0 / d, 0.0)).astype(BF16)
        c = jnp.sum(a, axis=0, keepdims=True)
        a2_ref[...] = (a * jnp.where(c > 0, 1.0 / c, 0.0)).astype(BF16)
        a1 = a1_ref[...]
        a2 = a2_ref[...]
        xv = xt_ref[...]
        t1 = _dot(a1, xv, 0, 0)
        t2 = 2.0 * _dot(a1, t1, 0, 0) - xv
        t3 = _mm(a2, xv)
        t4 = 2.0 * _mm(a2, t3) - xv
        txc = jnp.concatenate(
            [t.astype(BF16) for t in (xv, t1, t2, t3, t4)], axis=1)
        for q in range(4):
            nb = B // 4
            gq = _mm(txc, kg_ref[:, q * nb * 2 * U:(q + 1) * nb * 2 * U])
            cq = _mm(txc, kc_ref[:, q * nb * U:(q + 1) * nb * U])
            for j in range(nb):
                gx0_ref[q * nb + j] = (
                    gq[:, j * 2 * U:(j + 1) * 2 * U].astype(BF16))
                cx0_ref[q * nb + j] = cq[:, j * U:(j + 1) * U].astype(BF16)

    a1 = a1_ref[...]
    a2 = a2_ref[...]

    def diffuse(x0):
        x0 = x0.astype(BF16)
        t1 = _dot(a1, x0, 0, 0).astype(BF16)
        t2 = (2.0 * _dot(a1, t1, 0, 0) - x0.astype(F32)).astype(BF16)
        t3 = _mm(a2, x0).astype(BF16)
        t4 = (2.0 * _mm(a2, t3) - x0.astype(F32)).astype(BF16)
        return [x0, t1, t2, t3, t4]

    def cat(ts, j):
        return jnp.concatenate([t[:, j * U:(j + 1) * U] for t in ts], axis=1)

    H0w = h0_ref[...]
    H1w = h1_ref[...]

    Th = diffuse(H0w)
    r_list, u_list = [], []
    for j in range(BC):
        b = s * BC + j
        pre = _mm(cat(Th, j), wh0g_ref[...]) + gx0_ref[b] + bg0_ref[...]
        g = jax.nn.sigmoid(pre)
        r_list.append(g[:, :U])
        u_list.append(g[:, U:])
    RHw = jnp.concatenate(
        [r_list[j] * H0w[:, j * U:(j + 1) * U] for j in range(BC)], axis=1)
    Ch = diffuse(RHw)
    h0n_list = []
    for j in range(BC):
        b = s * BC + j
        pre = _mm(cat(Ch, j), wh0c_ref[...]) + cx0_ref[b] + bc0_ref[...]
        cv = jnp.tanh(pre)
        h0j = H0w[:, j * U:(j + 1) * U]
        h0n_list.append(u_list[j] * h0j + (1.0 - u_list[j]) * cv)
    H0nw = jnp.concatenate(h0n_list, axis=1)
    h0o_ref[...] = H0nw

    Ta = diffuse(H0nw)
    Tb = diffuse(H1w)
    ta_cat, r_list, u_list = [], [], []
    for j in range(BC):
        tc = cat(Ta, j)
        ta_cat.append(tc)
        pre = (_mm(tc, wa1g_ref[...]) + _mm(cat(Tb, j), wb1g_ref[...])
               + bg1_ref[...])
        g = jax.nn.sigmoid(pre)
        r_list.append(g[:, :U])
        u_list.append(g[:, U:])
    RHw = jnp.concatenate(
        [r_list[j] * H1w[:, j * U:(j + 1) * U] for j in range(BC)], axis=1)
    Ch = diffuse(RHw)
    h1n_list = []
    for j in range(BC):
        pre = (_mm(ta_cat[j], wa1c_ref[...]) + _mm(cat(Ch, j), wb1c_ref[...])
               + bc1_ref[...])
        cv = jnp.tanh(pre)
        h1j = H1w[:, j * U:(j + 1) * U]
        h1n_list.append(u_list[j] * h1j + (1.0 - u_list[j]) * cv)
    H1nw = jnp.concatenate(h1n_list, axis=1)
    h1o_ref[...] = H1nw

    out_ref[0] = _mm(H1nw, wpbd_ref[...]) + bp_ref[...]


def _msplit(w, d_x, d_h):
    w3 = w.reshape(d_x + d_h, M, -1)
    wx = w3[:d_x].transpose(1, 0, 2).reshape(M * d_x, -1)
    wh = w3[d_x:].transpose(1, 0, 2).reshape(M * d_h, -1)
    return wx, wh


def kernel(inputs, hidden_state, adj, W_gate_0, b_gate_0, W_cand_0, b_cand_0,
           W_gate_1, b_gate_1, W_cand_1, b_cand_1, W_proj, b_proj):
    h0 = hidden_state[0].reshape(B, N, U).transpose(1, 0, 2).reshape(N, B * U)
    h1 = hidden_state[1].reshape(B, N, U).transpose(1, 0, 2).reshape(N, B * U)
    xt = inputs.T
    wx0g, wh0g = _msplit(W_gate_0, 1, U)
    wx0c, wh0c = _msplit(W_cand_0, 1, U)
    wa1g, wb1g = _msplit(W_gate_1, U, U)
    wa1c, wb1c = _msplit(W_cand_1, U, U)
    wpbd = jnp.kron(jnp.eye(BC, dtype=F32), W_proj)
    eyeb = jnp.eye(B, dtype=F32)
    kg = jnp.concatenate([jnp.kron(eyeb, wx0g[m:m + 1]) for m in range(M)],
                         axis=0).astype(BF16)
    kc = jnp.concatenate([jnp.kron(eyeb, wx0c[m:m + 1]) for m in range(M)],
                         axis=0).astype(BF16)

    full = lambda *shape: pl.BlockSpec(shape, lambda b: (0,) * len(shape))
    wide = pl.BlockSpec((N, BC * U), lambda b: (0, b))
    out, h0o, h1o = pl.pallas_call(
        _dcgru_kernel,
        grid=(B // BC,),
        in_specs=[
            full(N, B),
            wide, wide,
            full(N, N),
            full(M * B, B * 2 * U), full(M * U, 2 * U), full(1, 2 * U),
            full(M * B, B * U), full(M * U, U), full(1, U),
            full(M * U, 2 * U), full(M * U, 2 * U), full(1, 2 * U),
            full(M * U, U), full(M * U, U), full(1, U),
            full(BC * U, BC), full(1, 1),
        ],
        out_specs=[
            pl.BlockSpec((1, N, BC), lambda b: (b, 0, 0)),
            wide, wide,
        ],
        out_shape=[
            jax.ShapeDtypeStruct((B // BC, N, BC), F32),
            jax.ShapeDtypeStruct((N, B * U), F32),
            jax.ShapeDtypeStruct((N, B * U), F32),
        ],
        scratch_shapes=[
            pltpu.VMEM((N, N), BF16),
            pltpu.VMEM((N, N), BF16),
            pltpu.VMEM((B, N, 2 * U), BF16),
            pltpu.VMEM((B, N, U), BF16),
        ],
    )(xt, h0, h1, adj,
      kg, wh0g, b_gate_0.reshape(1, -1), kc, wh0c, b_cand_0.reshape(1, -1),
      wa1g, wb1g, b_gate_1.reshape(1, -1), wa1c, wb1c, b_cand_1.reshape(1, -1),
      wpbd, b_proj.reshape(1, 1))

    output = out.transpose(0, 2, 1).reshape(B, N)
    hsl = [h.reshape(N, B, U).transpose(1, 0, 2).reshape(B, N * U)
           for h in (h0o, h1o)]
    return (output, jnp.stack(hsl))
